# baseline (speedup 1.0000x reference)
import jax
import jax.numpy as jnp
from jax.experimental import pallas as pl
from jax.experimental.pallas import tpu as pltpu

RATIO = 0.25
R = 0.3
K = 32

_N = 16384
_M = 4096
_GR = 128
_QR = _M // 128


def _fps_kernel(x_ref, y_ref, z_ref, qx_ref, qy_ref, qz_ref):
    xv = x_ref[...]
    yv = y_ref[...]
    zv = z_ref[...]
    row = jax.lax.broadcasted_iota(jnp.int32, (_GR, 128), 0)
    col = jax.lax.broadcasted_iota(jnp.int32, (_GR, 128), 1)
    idx2d = row * 128 + col
    qrow = jax.lax.broadcasted_iota(jnp.int32, (_QR, 128), 0)
    qcol = jax.lax.broadcasted_iota(jnp.int32, (_QR, 128), 1)
    qidx2d = qrow * 128 + qcol

    def body(i, state):
        dmin, cur = state
        mask = idx2d == cur
        cx = jnp.sum(jnp.where(mask, xv, 0.0))
        cy = jnp.sum(jnp.where(mask, yv, 0.0))
        cz = jnp.sum(jnp.where(mask, zv, 0.0))
        dx = xv - cx
        dy = yv - cy
        dz = zv - cz
        d = (dx * dx + dy * dy) + dz * dz
        dmin = jnp.minimum(dmin, d)
        mx = jnp.max(dmin)
        nxt = jnp.min(jnp.where(dmin == mx, idx2d, _N))
        qmask = qidx2d == i
        qx_ref[...] = jnp.where(qmask, cx, qx_ref[...])
        qy_ref[...] = jnp.where(qmask, cy, qy_ref[...])
        qz_ref[...] = jnp.where(qmask, cz, qz_ref[...])
        return dmin, nxt

    dmin0 = jnp.full((_GR, 128), jnp.inf, dtype=jnp.float32)
    jax.lax.fori_loop(0, _M, body, (dmin0, jnp.int32(0)))


def _fps_q(pos):
    x = pos[:, 0].reshape(_GR, 128)
    y = pos[:, 1].reshape(_GR, 128)
    z = pos[:, 2].reshape(_GR, 128)
    qx, qy, qz = pl.pallas_call(
        _fps_kernel,
        out_shape=[jax.ShapeDtypeStruct((_QR, 128), jnp.float32)] * 3,
    )(x, y, z)
    return jnp.stack(
        [qx.reshape(_M), qy.reshape(_M), qz.reshape(_M)], axis=-1
    )


_QB = 256


def _topk_kernel(q_ref, pT_ref, idx_ref, d2v_ref, d2_scr):
    qv = q_ref[...]
    pT = pT_ref[...]
    qn = jnp.sum(qv * qv, axis=1, keepdims=True)
    pn = jnp.sum(pT * pT, axis=0, keepdims=True)
    dot = jnp.dot(qv, pT, preferred_element_type=jnp.float32)
    d2_scr[...] = (qn + pn) - 2.0 * dot
    j2d = jax.lax.broadcasted_iota(jnp.int32, (_QB, _N), 1)
    for k in range(K):
        d2 = d2_scr[...]
        m = jnp.min(d2, axis=1, keepdims=True)
        cand = jnp.where(d2 == m, j2d, _N)
        isel = jnp.min(cand, axis=1, keepdims=True)
        idx_ref[:, k : k + 1] = isel
        d2v_ref[:, k : k + 1] = m
        d2_scr[...] = jnp.where(j2d == isel, jnp.inf, d2)


def _topk(q, posc):
    pT = posc.T
    nblk = _M // _QB
    idx, d2v = pl.pallas_call(
        _topk_kernel,
        grid=(nblk,),
        in_specs=[
            pl.BlockSpec((_QB, 3), lambda i: (i, 0)),
            pl.BlockSpec((3, _N), lambda i: (0, 0)),
        ],
        out_specs=[
            pl.BlockSpec((_QB, K), lambda i: (i, 0)),
            pl.BlockSpec((_QB, K), lambda i: (i, 0)),
        ],
        out_shape=[
            jax.ShapeDtypeStruct((_M, K), jnp.int32),
            jax.ShapeDtypeStruct((_M, K), jnp.float32),
        ],
        scratch_shapes=[pltpu.VMEM((_QB, _N), jnp.float32)],
    )(q, pT)
    return idx, d2v


def kernel(pos, batch, W1, b1, W2, b2, Wc, bc):
    posc = jax.lax.stop_gradient(pos)
    q = _fps_q(posc)
    idx, d2v = _topk(q, posc)
    valid = d2v <= (R * R)
    pos_i = q[:, None, :]
    pos_j = pos[idx]
    rel = pos_j - pos_i
    h = jax.nn.relu(rel @ W1 + b1) @ W2 + b2
    h = jnp.where(valid[:, :, None], h, -jnp.inf)
    x = jnp.max(h, axis=1)
    x = jnp.where(jnp.isfinite(x), x, 0.0)
    pooled = jnp.max(x, axis=0, keepdims=True)
    pooled = jnp.where(jnp.isfinite(pooled), pooled, 0.0)
    return pooled @ Wc + bc
